# baseline (speedup 1.0000x reference)
import jax
import jax.numpy as jnp
from jax.experimental import pallas as pl

ISIZE = 224
PSIZE = 30
ROW0 = 172
COL0 = 47
BB = 8


def _add_patch_kernel(x_ref, patch_ref, out_ref):
    out_ref[...] = x_ref[...]
    out_ref[:, :, ROW0:ROW0 + PSIZE, COL0:COL0 + PSIZE] = (
        out_ref[:, :, ROW0:ROW0 + PSIZE, COL0:COL0 + PSIZE] + patch_ref[...]
    )


def kernel(x, patch):
    batch = x.shape[0]
    grid = (batch // BB,)
    return pl.pallas_call(
        _add_patch_kernel,
        grid=grid,
        in_specs=[
            pl.BlockSpec((BB, 3, ISIZE, ISIZE), lambda i: (i, 0, 0, 0)),
            pl.BlockSpec((1, 3, PSIZE, PSIZE), lambda i: (0, 0, 0, 0)),
        ],
        out_specs=pl.BlockSpec((BB, 3, ISIZE, ISIZE), lambda i: (i, 0, 0, 0)),
        out_shape=jax.ShapeDtypeStruct(x.shape, x.dtype),
    )(x, patch)
